# Initial kernel scaffold; baseline (speedup 1.0000x reference)
import jax
import jax.numpy as jnp
from jax.experimental import pallas as pl


def kernel(x, edge_index, Wpre1, bpre1, Wpost1, bpost1, Wlin1, blin1, Wpre2, bpre2, Wpost2, bpost2, Wlin2, blin2, g1, b1, g2, b2, Wm1, bm1, Wm2, bm2):
    raise NotImplementedError("write your pallas kernel here")


# speedup vs baseline: 12.9282x; 12.9282x over previous
import functools
import math

import jax
import jax.numpy as jnp
from jax import lax
from jax.experimental import pallas as pl
from jax.experimental.pallas import tpu as pltpu
from jax.experimental.pallas import tpu_sc as plsc

N = 10000
E = 320000
F = 128
T = 4
TF = T * F
FOUT = 32
AVG_LOG = float(math.log(33.0))
NB = 1000
GRID_N = N // NB

_INTERPRET = False


def _pre_body(xn_ref, wt_ref, wb_ref, bpre_ref, a_ref, b_ref):
    xn = xn_ref[...]
    a_ref[...] = jnp.dot(xn, wt_ref[...], preferred_element_type=jnp.float32)
    b_ref[...] = jnp.dot(xn, wb_ref[...], preferred_element_type=jnp.float32) + bpre_ref[...]


def _pre_bn_body(h_ref, stats_ref, g_ref, b_ref, wt_ref, wb_ref, bpre_ref,
                 xn_ref, a_ref, b_out_ref):
    s = stats_ref[...]
    m = s[0:1, :] * (1.0 / N)
    v = s[1:2, :] * (1.0 / N) - m * m
    inv = jax.lax.rsqrt(v + 1e-5)
    xn = jnp.maximum((h_ref[...] - m) * inv * g_ref[...] + b_ref[...], 0.0)
    xn_ref[...] = xn
    a_ref[...] = jnp.dot(xn, wt_ref[...], preferred_element_type=jnp.float32)
    b_out_ref[...] = jnp.dot(xn, wb_ref[...], preferred_element_type=jnp.float32) + bpre_ref[...]


def _post_body(xn_ref, cnt_ref, sb_ref, qb_ref, mnb_ref, mxb_ref, a_ref,
               wp_ref, bpb_ref, wlin_ref, blin_ref,
               h_ref, stats_ref, acc_ref):
    i = pl.program_id(0)
    xn = xn_ref[...]
    a = a_ref[...]
    cnt = cnt_ref[...]
    c1 = jnp.maximum(cnt, 1.0)
    sb = sb_ref[...]
    s = cnt * a + sb
    mean = s / c1
    ssq = cnt * a * a + 2.0 * a * sb + qb_ref[...]
    var = ssq / c1 - mean * mean
    std = jnp.sqrt(jnp.maximum(var, 0.0) + 1e-5)
    has = cnt > 0.0
    mn = jnp.where(has, a + mnb_ref[...], 0.0)
    mx = jnp.where(has, a + mxb_ref[...], 0.0)
    lg = jnp.log(c1 + 1.0)
    amp_s = lg * (1.0 / AVG_LOG)
    att_s = AVG_LOG / lg
    outs = []
    for t in range(T):
        sl = slice(t * F, (t + 1) * F)
        agg_t = jnp.concatenate([mean[:, sl], mn[:, sl], mx[:, sl], std[:, sl]], axis=1)
        pin = jnp.concatenate([xn, amp_s * agg_t, att_s * agg_t, agg_t], axis=1)
        outs.append(jnp.dot(pin, wp_ref[t], preferred_element_type=jnp.float32))
    flat = jnp.concatenate(outs, axis=1) + bpb_ref[...]
    h = jnp.dot(flat, wlin_ref[...], preferred_element_type=jnp.float32) + blin_ref[...]
    h_ref[...] = h

    @pl.when(i == 0)
    def _():
        acc_ref[...] = jnp.zeros_like(acc_ref)

    acc_ref[0:1, :] += jnp.sum(h, axis=0, keepdims=True)
    acc_ref[1:2, :] += jnp.sum(h * h, axis=0, keepdims=True)

    @pl.when(i == GRID_N - 1)
    def _():
        stats_ref[...] = acc_ref[...]


def _head_body(h_ref, stats_ref, g_ref, b_ref, wm1_ref, bm1_ref, wm2_ref, bm2_ref, out_ref):
    s = stats_ref[...]
    m = s[0:1, :] * (1.0 / N)
    v = s[1:2, :] * (1.0 / N) - m * m
    inv = jax.lax.rsqrt(v + 1e-5)
    xn = jnp.maximum((h_ref[...] - m) * inv * g_ref[...] + b_ref[...], 0.0)
    y = jnp.maximum(jnp.dot(xn, wm1_ref[...], preferred_element_type=jnp.float32) + bm1_ref[...], 0.0)
    out_ref[...] = jnp.dot(y, wm2_ref[...], preferred_element_type=jnp.float32) + bm2_ref[...]


def _row_spec(cols):
    return pl.BlockSpec((NB, cols), lambda i: (i, 0))


def _full_spec(shape):
    nd = len(shape)
    return pl.BlockSpec(shape, lambda i: (0,) * nd)


def _pre_call(xn, Wt, Wb, bpre2):
    return pl.pallas_call(
        _pre_body,
        grid=(GRID_N,),
        in_specs=[_row_spec(F), _full_spec((F, TF)), _full_spec((F, TF)), _full_spec((1, TF))],
        out_specs=[_row_spec(TF), _row_spec(TF)],
        out_shape=[jax.ShapeDtypeStruct((N, TF), jnp.float32)] * 2,
        interpret=_INTERPRET,
    )(xn, Wt, Wb, bpre2)


def _pre_bn_call(h, stats, g, b, Wt, Wb, bpre2):
    return pl.pallas_call(
        _pre_bn_body,
        grid=(GRID_N,),
        in_specs=[_row_spec(F), _full_spec((2, F)), _full_spec((1, F)), _full_spec((1, F)),
                  _full_spec((F, TF)), _full_spec((F, TF)), _full_spec((1, TF))],
        out_specs=[_row_spec(F), _row_spec(TF), _row_spec(TF)],
        out_shape=[jax.ShapeDtypeStruct((N, F), jnp.float32),
                   jax.ShapeDtypeStruct((N, TF), jnp.float32),
                   jax.ShapeDtypeStruct((N, TF), jnp.float32)],
        interpret=_INTERPRET,
    )(h, stats, g, b, Wt, Wb, bpre2)


def _post_call(xn, cnt, SB, QB, mnB, mxB, A, Wp, bpb, Wlin, blin):
    return pl.pallas_call(
        _post_body,
        grid=(GRID_N,),
        in_specs=[_row_spec(F), _row_spec(1), _row_spec(TF), _row_spec(TF),
                  _row_spec(TF), _row_spec(TF), _row_spec(TF),
                  _full_spec((T, 13 * F, FOUT)),
                  _full_spec((1, F)), _full_spec((F, F)), _full_spec((1, F))],
        out_specs=[_row_spec(F), _full_spec((2, F))],
        out_shape=[jax.ShapeDtypeStruct((N, F), jnp.float32),
                   jax.ShapeDtypeStruct((2, F), jnp.float32)],
        scratch_shapes=[pltpu.VMEM((2, F), jnp.float32)],
        interpret=_INTERPRET,
    )(xn, cnt, SB, QB, mnB, mxB, A, Wp, bpb, Wlin, blin)


def _head_call(h, stats, g, b, Wm1, bm1, Wm2, bm2):
    return pl.pallas_call(
        _head_body,
        grid=(GRID_N,),
        in_specs=[_row_spec(F), _full_spec((2, F)), _full_spec((1, F)), _full_spec((1, F)),
                  _full_spec((F, F)), _full_spec((1, F)), _full_spec((F, F)), _full_spec((1, F))],
        out_specs=_row_spec(F),
        out_shape=jax.ShapeDtypeStruct((N, F), jnp.float32),
        interpret=_INTERPRET,
    )(h, stats, g, b, Wm1, bm1, Wm2, bm2)


def _edge_phase(B, src, dst):
    Bs = B[src]
    ones = jnp.ones((E,), jnp.float32)
    cnt = jax.ops.segment_sum(ones, dst, num_segments=N)
    SB = jax.ops.segment_sum(Bs, dst, num_segments=N)
    QB = jax.ops.segment_sum(Bs * Bs, dst, num_segments=N)
    mnB = jax.ops.segment_min(Bs, dst, num_segments=N)
    mxB = jax.ops.segment_max(Bs, dst, num_segments=N)
    has = (cnt > 0.0)[:, None]
    mnB = jnp.where(has, mnB, 0.0)
    mxB = jnp.where(has, mxB, 0.0)
    return cnt[:, None], SB, QB, mnB, mxB


def _prep_weights(Wpre, bpre, Wpost, bpost):
    Wt = Wpre[:, :F, :].transpose(1, 0, 2).reshape(F, TF)
    Wb = Wpre[:, F:, :].transpose(1, 0, 2).reshape(F, TF)
    bpre2 = bpre.reshape(1, TF)
    bpb = bpost.reshape(1, T * FOUT)
    return Wt, Wb, bpre2, Wpost, bpb


def kernel(x, edge_index, Wpre1, bpre1, Wpost1, bpost1, Wlin1, blin1,
           Wpre2, bpre2, Wpost2, bpost2, Wlin2, blin2,
           g1, b1, g2, b2, Wm1, bm1, Wm2, bm2):
    src = edge_index[0]
    dst = edge_index[1]

    Wt1, Wb1, bp1, Wp1, bpb1 = _prep_weights(Wpre1, bpre1, Wpost1, bpost1)
    Wt2, Wb2, bp2, Wp2, bpb2 = _prep_weights(Wpre2, bpre2, Wpost2, bpost2)

    A1, B1 = _pre_call(x, Wt1, Wb1, bp1)
    cnt, SB, QB, mnB, mxB = _edge_phase(B1, src, dst)
    h1, stats1 = _post_call(x, cnt, SB, QB, mnB, mxB, A1,
                            Wp1, bpb1, Wlin1, blin1.reshape(1, F))

    xn2, A2, B2 = _pre_bn_call(h1, stats1, g1.reshape(1, F), b1.reshape(1, F), Wt2, Wb2, bp2)
    cnt2, SB2, QB2, mnB2, mxB2 = _edge_phase(B2, src, dst)
    h2, stats2 = _post_call(xn2, cnt2, SB2, QB2, mnB2, mxB2, A2,
                            Wp2, bpb2, Wlin2, blin2.reshape(1, F))

    return _head_call(h2, stats2, g2.reshape(1, F), b2.reshape(1, F),
                      Wm1, bm1.reshape(1, F), Wm2, bm2.reshape(1, F))


# speedup vs baseline: 24.2950x; 1.8792x over previous
import functools
import math

import jax
import jax.numpy as jnp
from jax import lax
from jax.experimental import pallas as pl
from jax.experimental.pallas import tpu as pltpu
from jax.experimental.pallas import tpu_sc as plsc

N = 10000
E = 320000
F = 128
T = 4
TF = T * F
FOUT = 32
AVG_LOG = float(math.log(33.0))
NB = 1000
GRID_N = N // NB

_INTERPRET = False


def _pre_body(xn_ref, wt_ref, wb_ref, bpre_ref, a_ref, b3_ref):
    xn = xn_ref[...]
    a_ref[...] = jnp.dot(xn, wt_ref[...], preferred_element_type=jnp.float32)
    b = jnp.dot(xn, wb_ref[...], preferred_element_type=jnp.float32) + bpre_ref[...]
    for c in range(NCH):
        b3_ref[c] = b[:, c * FC:(c + 1) * FC]


def _pre_bn_body(h_ref, stats_ref, g_ref, b_ref, wt_ref, wb_ref, bpre_ref,
                 xn_ref, a_ref, b3_ref):
    s = stats_ref[...]
    m = s[0:1, :] * (1.0 / N)
    v = s[1:2, :] * (1.0 / N) - m * m
    inv = jax.lax.rsqrt(v + 1e-5)
    xn = jnp.maximum((h_ref[...] - m) * inv * g_ref[...] + b_ref[...], 0.0)
    xn_ref[...] = xn
    a_ref[...] = jnp.dot(xn, wt_ref[...], preferred_element_type=jnp.float32)
    b = jnp.dot(xn, wb_ref[...], preferred_element_type=jnp.float32) + bpre_ref[...]
    for c in range(NCH):
        b3_ref[c] = b[:, c * FC:(c + 1) * FC]


def _post_body(xn_ref, cnt_ref, sb_ref, qb_ref, mnb_ref, mxb_ref, a_ref,
               wp_ref, bpb_ref, wlin_ref, blin_ref,
               h_ref, stats_ref, acc_ref):
    i = pl.program_id(0)
    xn = xn_ref[...]
    a = a_ref[...]
    cnt = cnt_ref[...]
    c1 = jnp.maximum(cnt, 1.0)
    sb = jnp.concatenate([sb_ref[c] for c in range(NCH)], axis=-1)
    qb = jnp.concatenate([qb_ref[c] for c in range(NCH)], axis=-1)
    mnb = jnp.concatenate([mnb_ref[c] for c in range(NCH)], axis=-1)
    mxb = jnp.concatenate([mxb_ref[c] for c in range(NCH)], axis=-1)
    s = cnt * a + sb
    mean = s / c1
    ssq = cnt * a * a + 2.0 * a * sb + qb
    var = ssq / c1 - mean * mean
    std = jnp.sqrt(jnp.maximum(var, 0.0) + 1e-5)
    has = cnt > 0.0
    mn = jnp.where(has, a + mnb, 0.0)
    mx = jnp.where(has, a + mxb, 0.0)
    lg = jnp.log(c1 + 1.0)
    amp_s = lg * (1.0 / AVG_LOG)
    att_s = AVG_LOG / lg
    outs = []
    for t in range(T):
        sl = slice(t * F, (t + 1) * F)
        agg_t = jnp.concatenate([mean[:, sl], mn[:, sl], mx[:, sl], std[:, sl]], axis=1)
        pin = jnp.concatenate([xn, amp_s * agg_t, att_s * agg_t, agg_t], axis=1)
        outs.append(jnp.dot(pin, wp_ref[t], preferred_element_type=jnp.float32))
    flat = jnp.concatenate(outs, axis=1) + bpb_ref[...]
    h = jnp.dot(flat, wlin_ref[...], preferred_element_type=jnp.float32) + blin_ref[...]
    h_ref[...] = h

    @pl.when(i == 0)
    def _():
        acc_ref[...] = jnp.zeros_like(acc_ref)

    acc_ref[0:1, :] += jnp.sum(h, axis=0, keepdims=True)
    acc_ref[1:2, :] += jnp.sum(h * h, axis=0, keepdims=True)

    @pl.when(i == GRID_N - 1)
    def _():
        stats_ref[...] = acc_ref[...]


def _head_body(h_ref, stats_ref, g_ref, b_ref, wm1_ref, bm1_ref, wm2_ref, bm2_ref, out_ref):
    s = stats_ref[...]
    m = s[0:1, :] * (1.0 / N)
    v = s[1:2, :] * (1.0 / N) - m * m
    inv = jax.lax.rsqrt(v + 1e-5)
    xn = jnp.maximum((h_ref[...] - m) * inv * g_ref[...] + b_ref[...], 0.0)
    y = jnp.maximum(jnp.dot(xn, wm1_ref[...], preferred_element_type=jnp.float32) + bm1_ref[...], 0.0)
    out_ref[...] = jnp.dot(y, wm2_ref[...], preferred_element_type=jnp.float32) + bm2_ref[...]


def _row_spec(cols):
    return pl.BlockSpec((NB, cols), lambda i: (i, 0))


def _chunk_spec():
    return pl.BlockSpec((NCH, NB, FC), lambda i: (0, i, 0))


def _full_spec(shape):
    nd = len(shape)
    return pl.BlockSpec(shape, lambda i: (0,) * nd)


def _pre_call(xn, Wt, Wb, bpre2):
    return pl.pallas_call(
        _pre_body,
        grid=(GRID_N,),
        in_specs=[_row_spec(F), _full_spec((F, TF)), _full_spec((F, TF)), _full_spec((1, TF))],
        out_specs=[_row_spec(TF), _chunk_spec()],
        out_shape=[jax.ShapeDtypeStruct((N, TF), jnp.float32),
                   jax.ShapeDtypeStruct((NCH, N, FC), jnp.float32)],
        interpret=_INTERPRET,
    )(xn, Wt, Wb, bpre2)


def _pre_bn_call(h, stats, g, b, Wt, Wb, bpre2):
    return pl.pallas_call(
        _pre_bn_body,
        grid=(GRID_N,),
        in_specs=[_row_spec(F), _full_spec((2, F)), _full_spec((1, F)), _full_spec((1, F)),
                  _full_spec((F, TF)), _full_spec((F, TF)), _full_spec((1, TF))],
        out_specs=[_row_spec(F), _row_spec(TF), _chunk_spec()],
        out_shape=[jax.ShapeDtypeStruct((N, F), jnp.float32),
                   jax.ShapeDtypeStruct((N, TF), jnp.float32),
                   jax.ShapeDtypeStruct((NCH, N, FC), jnp.float32)],
        interpret=_INTERPRET,
    )(h, stats, g, b, Wt, Wb, bpre2)


def _post_call(xn, cnt, SB, QB, mnB, mxB, A, Wp, bpb, Wlin, blin):
    return pl.pallas_call(
        _post_body,
        grid=(GRID_N,),
        in_specs=[_row_spec(F), _row_spec(1), _chunk_spec(), _chunk_spec(),
                  _chunk_spec(), _chunk_spec(), _row_spec(TF),
                  _full_spec((T, 13 * F, FOUT)),
                  _full_spec((1, F)), _full_spec((F, F)), _full_spec((1, F))],
        out_specs=[_row_spec(F), _full_spec((2, F))],
        out_shape=[jax.ShapeDtypeStruct((N, F), jnp.float32),
                   jax.ShapeDtypeStruct((2, F), jnp.float32)],
        scratch_shapes=[pltpu.VMEM((2, F), jnp.float32)],
        interpret=_INTERPRET,
    )(xn, cnt, SB, QB, mnB, mxB, A, Wp, bpb, Wlin, blin)


def _head_call(h, stats, g, b, Wm1, bm1, Wm2, bm2):
    return pl.pallas_call(
        _head_body,
        grid=(GRID_N,),
        in_specs=[_row_spec(F), _full_spec((2, F)), _full_spec((1, F)), _full_spec((1, F)),
                  _full_spec((F, F)), _full_spec((1, F)), _full_spec((F, F)), _full_spec((1, F))],
        out_specs=_row_spec(F),
        out_shape=jax.ShapeDtypeStruct((N, F), jnp.float32),
        interpret=_INTERPRET,
    )(h, stats, g, b, Wm1, bm1, Wm2, bm2)


NWK = 32
NPW = 320
NPAD = NWK * NPW
QE = 256
EP = E + 2048
FC = 64
NCH = TF // FC
_BIG = 3.0e38

def _get_mesh():
    return plsc.VectorSubcoreMesh(core_axis_name="c", subcore_axis_name="s",
                                  num_cores=2, num_subcores=16)


def _wid():
    return lax.axis_index("s") * 2 + lax.axis_index("c")


def _vgather(x, idx):
    dn = lax.GatherDimensionNumbers(offset_dims=(), collapsed_slice_dims=(0,),
                                    start_index_map=(0,))
    return lax.gather(x, idx[:, None], dimension_numbers=dn, slice_sizes=(1,),
                      mode=lax.GatherScatterMode.PROMISE_IN_BOUNDS)


_CH = 2000


def _prepass_body(srce_ref, dste_ref, packed_ref, nq_ref, src_v, dst_v, stage_v, tmp_v):
    w = _wid()
    lo = w * NPW
    hi = lo + NPW
    dummy = jnp.full((16,), NPW << 16, jnp.int32)

    def chunk_body(ci, carry):
        cbase = pl.multiple_of(ci * _CH, 8)
        pltpu.sync_copy(srce_ref.at[pl.ds(cbase, _CH)], src_v)
        pltpu.sync_copy(dste_ref.at[pl.ds(cbase, _CH)], dst_v)

        def grp(gi, c2):
            cnt, off = c2
            s = src_v[pl.ds(gi * 16, 16)]
            dvec = dst_v[pl.ds(gi * 16, 16)]
            m = (dvec >= lo) & (dvec < hi)
            pk = s | ((dvec - lo) << 16)
            mi = m.astype(jnp.int32)
            pos = cnt + plsc.cumsum(mi) - 1
            plsc.store_scatter(stage_v, [pos], pk, mask=m)
            cnt = cnt + jnp.sum(mi)

            def do_flush(co):
                c, o = co
                pltpu.sync_copy(stage_v.at[pl.ds(0, 1024)],
                                packed_ref.at[pl.ds(pl.multiple_of(w * EP + o, 8), 1024)])
                stage_v[pl.ds(0, 16)] = stage_v[pl.ds(1024, 16)]
                stage_v[pl.ds(16, 16)] = stage_v[pl.ds(1040, 16)]
                return (c - 1024, o + 1024)

            return lax.cond(cnt >= 1024, do_flush, lambda co: co, (cnt, off))

        return lax.fori_loop(0, _CH // 16, grp, carry)

    cnt, off = lax.fori_loop(0, E // _CH, chunk_body, (0, 0))

    stage_v[pl.ds(cnt, 16)] = dummy
    cnt = (cnt + 15) & -16

    def pad16(c):
        stage_v[pl.ds(c, 16)] = dummy
        return c + 16

    cnt = lax.while_loop(lambda c: lax.rem(c, QE) != 0, pad16, cnt)

    def fflush(qi, _):
        pltpu.sync_copy(stage_v.at[pl.ds(pl.multiple_of(qi * QE, 8), QE)],
                        packed_ref.at[pl.ds(pl.multiple_of(w * EP + off + qi * QE, 8), QE)])
        return 0

    lax.fori_loop(0, cnt // QE, fflush, 0)
    tmp_v[...] = jnp.broadcast_to((off + cnt) // QE, (16,)).astype(jnp.int32)
    pltpu.sync_copy(tmp_v, nq_ref.at[pl.ds(pl.multiple_of(w * 16, 8), 16)])


def _prepass(srce, dste):
    f = pl.kernel(
        _prepass_body,
        out_type=[jax.ShapeDtypeStruct((NWK * EP,), jnp.int32),
                  jax.ShapeDtypeStruct((NWK * 16,), jnp.int32)],
        mesh=_get_mesh(),
        compiler_params=pltpu.CompilerParams(needs_layout_passes=False,
                                             use_tc_tiling_on_sc=False),
        scratch_types=[pltpu.VMEM((_CH,), jnp.int32),
                       pltpu.VMEM((_CH,), jnp.int32),
                       pltpu.VMEM((1312,), jnp.int32),
                       pltpu.VMEM((16,), jnp.int32)],
    )
    return f(srce, dste)


_IOTA = tuple(range(16))


def _stats_body(packed_ref, nq_ref, tab_ref, s3_ref, q3_ref, mn3_ref, mx3_ref,
                cnt_ref, nq_v, pk_v, src_v, rows_v, accS, accQ, accMN, accMX,
                accC, sem):
    w = _wid()
    iota = jnp.arange(16, dtype=jnp.int32)
    onehot = (iota == 0).astype(jnp.float32)
    pltpu.sync_copy(nq_ref.at[pl.ds(pl.multiple_of(w * 16, 8), 16)], nq_v)
    nq = jnp.max(nq_v[...])

    def chunk(c, _):
        def initacc(i, __):
            z = jnp.zeros((16,), jnp.float32)
            for g in range(FC // 16):
                accS[i, pl.ds(g * 16, 16)] = z
                accQ[i, pl.ds(g * 16, 16)] = z
                accMN[i, pl.ds(g * 16, 16)] = z + _BIG
                accMX[i, pl.ds(g * 16, 16)] = z - _BIG
            return 0

        lax.fori_loop(0, NPW + 1, initacc, 0, unroll=4)

        @pl.when(c == 0)
        def _():
            def initc(i, __):
                accC[pl.ds(i * 16, 16)] = jnp.zeros((16,), jnp.float32)
                return 0
            lax.fori_loop(0, (NPW + 16) // 16, initc, 0)

        def quantum(qi, __):
            pltpu.sync_copy(packed_ref.at[pl.ds(pl.multiple_of(w * EP + qi * QE, 8), QE)], pk_v)
            for k in range(QE // 16):
                src_v[pl.ds(k * 16, 16)] = pk_v[pl.ds(k * 16, 16)] & 0xFFFF
            pltpu.async_copy(tab_ref.at[c].at[src_v], rows_v, sem).wait()

            def grp(g, ___):
                dl_v = pk_v[pl.ds(g * 16, 16)] >> 16
                for l in range(16):
                    dl = _vgather(dl_v, jnp.full((16,), l, jnp.int32))
                    j = g * 16 + l
                    plsc.addupdate_scatter(accC, [dl + iota], onehot)
                    for gg in range(FC // 16):
                        col = iota + (gg * 16)
                        v = rows_v[j, pl.ds(gg * 16, 16)]
                        plsc.addupdate_scatter(accS, [dl, col], v)
                        plsc.addupdate_scatter(accQ, [dl, col], v * v)
                        mn = plsc.load_gather(accMN, [dl, col])
                        plsc.store_scatter(accMN, [dl, col], jnp.minimum(mn, v))
                        mx = plsc.load_gather(accMX, [dl, col])
                        plsc.store_scatter(accMX, [dl, col], jnp.maximum(mx, v))
                return 0

            lax.fori_loop(0, QE // 16, grp, 0)
            return 0

        lax.fori_loop(0, nq, quantum, 0)

        wbase = pl.multiple_of(w * NPW, 8)
        pltpu.sync_copy(accS.at[pl.ds(0, NPW)], s3_ref.at[c, pl.ds(wbase, NPW)])
        pltpu.sync_copy(accQ.at[pl.ds(0, NPW)], q3_ref.at[c, pl.ds(wbase, NPW)])
        pltpu.sync_copy(accMN.at[pl.ds(0, NPW)], mn3_ref.at[c, pl.ds(wbase, NPW)])
        pltpu.sync_copy(accMX.at[pl.ds(0, NPW)], mx3_ref.at[c, pl.ds(wbase, NPW)])

        @pl.when(c == NCH - 1)
        def _():
            def scale(i, __):
                accC[pl.ds(i * 16, 16)] = accC[pl.ds(i * 16, 16)] * (1.0 / NCH)
                return 0
            lax.fori_loop(0, NPW // 16, scale, 0)
            pltpu.sync_copy(accC.at[pl.ds(0, NPW)], cnt_ref.at[pl.ds(pl.multiple_of(w * NPW, 8), NPW)])

        return 0

    lax.fori_loop(0, NCH, chunk, 0)


def _stats(packed, nqarr, tab):
    f = pl.kernel(
        _stats_body,
        out_type=[jax.ShapeDtypeStruct((NCH, NPAD, FC), jnp.float32),
                  jax.ShapeDtypeStruct((NCH, NPAD, FC), jnp.float32),
                  jax.ShapeDtypeStruct((NCH, NPAD, FC), jnp.float32),
                  jax.ShapeDtypeStruct((NCH, NPAD, FC), jnp.float32),
                  jax.ShapeDtypeStruct((NWK * NPW,), jnp.float32)],
        mesh=_get_mesh(),
        compiler_params=pltpu.CompilerParams(needs_layout_passes=False,
                                             use_tc_tiling_on_sc=False),
        scratch_types=[pltpu.VMEM((16,), jnp.int32),
                       pltpu.VMEM((QE,), jnp.int32),
                       pltpu.VMEM((QE,), jnp.int32),
                       pltpu.VMEM((QE, FC), jnp.float32),
                       pltpu.VMEM((NPW + 1, FC), jnp.float32),
                       pltpu.VMEM((NPW + 1, FC), jnp.float32),
                       pltpu.VMEM((NPW + 1, FC), jnp.float32),
                       pltpu.VMEM((NPW + 1, FC), jnp.float32),
                       pltpu.VMEM((NPW + 16, ), jnp.float32),
                       pltpu.SemaphoreType.DMA],
    )
    return f(packed, nqarr, tab)


def _edge_phase_sc(tab, packed, nqarr):
    S3, Q3, MN3, MX3, cnt1 = _stats(packed, nqarr, tab)
    cnt = cnt1[:N].reshape(N, 1)
    return cnt, S3, Q3, MN3, MX3


def _prep_weights(Wpre, bpre, Wpost, bpost):
    Wt = Wpre[:, :F, :].transpose(1, 0, 2).reshape(F, TF)
    Wb = Wpre[:, F:, :].transpose(1, 0, 2).reshape(F, TF)
    bpre2 = bpre.reshape(1, TF)
    bpb = bpost.reshape(1, T * FOUT)
    return Wt, Wb, bpre2, Wpost, bpb


def kernel(x, edge_index, Wpre1, bpre1, Wpost1, bpost1, Wlin1, blin1,
           Wpre2, bpre2, Wpost2, bpost2, Wlin2, blin2,
           g1, b1, g2, b2, Wm1, bm1, Wm2, bm2):
    src = edge_index[0]
    dst = edge_index[1]

    Wt1, Wb1, bp1, Wp1, bpb1 = _prep_weights(Wpre1, bpre1, Wpost1, bpost1)
    Wt2, Wb2, bp2, Wp2, bpb2 = _prep_weights(Wpre2, bpre2, Wpost2, bpost2)

    packed, nqarr = _prepass(src, dst)

    A1, B1 = _pre_call(x, Wt1, Wb1, bp1)
    cnt, SB, QB, mnB, mxB = _edge_phase_sc(B1, packed, nqarr)
    h1, stats1 = _post_call(x, cnt, SB, QB, mnB, mxB, A1,
                            Wp1, bpb1, Wlin1, blin1.reshape(1, F))

    xn2, A2, B2 = _pre_bn_call(h1, stats1, g1.reshape(1, F), b1.reshape(1, F), Wt2, Wb2, bp2)
    cnt2, SB2, QB2, mnB2, mxB2 = _edge_phase_sc(B2, packed, nqarr)
    h2, stats2 = _post_call(xn2, cnt2, SB2, QB2, mnB2, mxB2, A2,
                            Wp2, bpb2, Wlin2, blin2.reshape(1, F))

    return _head_call(h2, stats2, g2.reshape(1, F), b2.reshape(1, F),
                      Wm1, bm1.reshape(1, F), Wm2, bm2.reshape(1, F))


# speedup vs baseline: 25.4516x; 1.0476x over previous
import functools
import math

import jax
import jax.numpy as jnp
from jax import lax
from jax.experimental import pallas as pl
from jax.experimental.pallas import tpu as pltpu
from jax.experimental.pallas import tpu_sc as plsc

N = 10000
E = 320000
F = 128
T = 4
TF = T * F
FOUT = 32
AVG_LOG = float(math.log(33.0))
NB = 1000
GRID_N = N // NB

_INTERPRET = False


def _pre_body(xn_ref, wt_ref, wb_ref, bpre_ref, a_ref, b3_ref):
    xn = xn_ref[...]
    a_ref[...] = jnp.dot(xn, wt_ref[...], preferred_element_type=jnp.float32)
    b = jnp.dot(xn, wb_ref[...], preferred_element_type=jnp.float32) + bpre_ref[...]
    for c in range(NCH):
        b3_ref[c] = b[:, c * FC:(c + 1) * FC]


def _pre_bn_body(h_ref, stats_ref, g_ref, b_ref, wt_ref, wb_ref, bpre_ref,
                 xn_ref, a_ref, b3_ref):
    s = stats_ref[...]
    m = s[0:1, :] * (1.0 / N)
    v = s[1:2, :] * (1.0 / N) - m * m
    inv = jax.lax.rsqrt(v + 1e-5)
    xn = jnp.maximum((h_ref[...] - m) * inv * g_ref[...] + b_ref[...], 0.0)
    xn_ref[...] = xn
    a_ref[...] = jnp.dot(xn, wt_ref[...], preferred_element_type=jnp.float32)
    b = jnp.dot(xn, wb_ref[...], preferred_element_type=jnp.float32) + bpre_ref[...]
    for c in range(NCH):
        b3_ref[c] = b[:, c * FC:(c + 1) * FC]


def _post_body(xn_ref, cnt_ref, sb_ref, qb_ref, mnb_ref, mxb_ref, a_ref,
               wp_ref, bpb_ref, wlin_ref, blin_ref,
               h_ref, stats_ref, acc_ref):
    i = pl.program_id(0)
    xn = xn_ref[...]
    a = a_ref[...]
    cnt = cnt_ref[...]
    c1 = jnp.maximum(cnt, 1.0)
    sb = jnp.concatenate([sb_ref[c] for c in range(NCH)], axis=-1)
    qb = jnp.concatenate([qb_ref[c] for c in range(NCH)], axis=-1)
    mnb = jnp.concatenate([mnb_ref[c] for c in range(NCH)], axis=-1)
    mxb = jnp.concatenate([mxb_ref[c] for c in range(NCH)], axis=-1)
    s = cnt * a + sb
    mean = s / c1
    ssq = cnt * a * a + 2.0 * a * sb + qb
    var = ssq / c1 - mean * mean
    std = jnp.sqrt(jnp.maximum(var, 0.0) + 1e-5)
    has = cnt > 0.0
    mn = jnp.where(has, a + mnb, 0.0)
    mx = jnp.where(has, a + mxb, 0.0)
    lg = jnp.log(c1 + 1.0)
    amp_s = lg * (1.0 / AVG_LOG)
    att_s = AVG_LOG / lg
    outs = []
    for t in range(T):
        sl = slice(t * F, (t + 1) * F)
        agg_t = jnp.concatenate([mean[:, sl], mn[:, sl], mx[:, sl], std[:, sl]], axis=1)
        pin = jnp.concatenate([xn, amp_s * agg_t, att_s * agg_t, agg_t], axis=1)
        outs.append(jnp.dot(pin, wp_ref[t], preferred_element_type=jnp.float32))
    flat = jnp.concatenate(outs, axis=1) + bpb_ref[...]
    h = jnp.dot(flat, wlin_ref[...], preferred_element_type=jnp.float32) + blin_ref[...]
    h_ref[...] = h

    @pl.when(i == 0)
    def _():
        acc_ref[...] = jnp.zeros_like(acc_ref)

    acc_ref[0:1, :] += jnp.sum(h, axis=0, keepdims=True)
    acc_ref[1:2, :] += jnp.sum(h * h, axis=0, keepdims=True)

    @pl.when(i == GRID_N - 1)
    def _():
        stats_ref[...] = acc_ref[...]


def _head_body(h_ref, stats_ref, g_ref, b_ref, wm1_ref, bm1_ref, wm2_ref, bm2_ref, out_ref):
    s = stats_ref[...]
    m = s[0:1, :] * (1.0 / N)
    v = s[1:2, :] * (1.0 / N) - m * m
    inv = jax.lax.rsqrt(v + 1e-5)
    xn = jnp.maximum((h_ref[...] - m) * inv * g_ref[...] + b_ref[...], 0.0)
    y = jnp.maximum(jnp.dot(xn, wm1_ref[...], preferred_element_type=jnp.float32) + bm1_ref[...], 0.0)
    out_ref[...] = jnp.dot(y, wm2_ref[...], preferred_element_type=jnp.float32) + bm2_ref[...]


def _row_spec(cols):
    return pl.BlockSpec((NB, cols), lambda i: (i, 0))


def _chunk_spec():
    return pl.BlockSpec((NCH, NB, FC), lambda i: (0, i, 0))


def _full_spec(shape):
    nd = len(shape)
    return pl.BlockSpec(shape, lambda i: (0,) * nd)


def _pre_call(xn, Wt, Wb, bpre2):
    return pl.pallas_call(
        _pre_body,
        grid=(GRID_N,),
        in_specs=[_row_spec(F), _full_spec((F, TF)), _full_spec((F, TF)), _full_spec((1, TF))],
        out_specs=[_row_spec(TF), _chunk_spec()],
        out_shape=[jax.ShapeDtypeStruct((N, TF), jnp.float32),
                   jax.ShapeDtypeStruct((NCH, N, FC), jnp.float32)],
        interpret=_INTERPRET,
    )(xn, Wt, Wb, bpre2)


def _pre_bn_call(h, stats, g, b, Wt, Wb, bpre2):
    return pl.pallas_call(
        _pre_bn_body,
        grid=(GRID_N,),
        in_specs=[_row_spec(F), _full_spec((2, F)), _full_spec((1, F)), _full_spec((1, F)),
                  _full_spec((F, TF)), _full_spec((F, TF)), _full_spec((1, TF))],
        out_specs=[_row_spec(F), _row_spec(TF), _chunk_spec()],
        out_shape=[jax.ShapeDtypeStruct((N, F), jnp.float32),
                   jax.ShapeDtypeStruct((N, TF), jnp.float32),
                   jax.ShapeDtypeStruct((NCH, N, FC), jnp.float32)],
        interpret=_INTERPRET,
    )(h, stats, g, b, Wt, Wb, bpre2)


def _post_call(xn, cnt, SB, QB, mnB, mxB, A, Wp, bpb, Wlin, blin):
    return pl.pallas_call(
        _post_body,
        grid=(GRID_N,),
        in_specs=[_row_spec(F), _row_spec(1), _chunk_spec(), _chunk_spec(),
                  _chunk_spec(), _chunk_spec(), _row_spec(TF),
                  _full_spec((T, 13 * F, FOUT)),
                  _full_spec((1, F)), _full_spec((F, F)), _full_spec((1, F))],
        out_specs=[_row_spec(F), _full_spec((2, F))],
        out_shape=[jax.ShapeDtypeStruct((N, F), jnp.float32),
                   jax.ShapeDtypeStruct((2, F), jnp.float32)],
        scratch_shapes=[pltpu.VMEM((2, F), jnp.float32)],
        interpret=_INTERPRET,
    )(xn, cnt, SB, QB, mnB, mxB, A, Wp, bpb, Wlin, blin)


def _head_call(h, stats, g, b, Wm1, bm1, Wm2, bm2):
    return pl.pallas_call(
        _head_body,
        grid=(GRID_N,),
        in_specs=[_row_spec(F), _full_spec((2, F)), _full_spec((1, F)), _full_spec((1, F)),
                  _full_spec((F, F)), _full_spec((1, F)), _full_spec((F, F)), _full_spec((1, F))],
        out_specs=_row_spec(F),
        out_shape=jax.ShapeDtypeStruct((N, F), jnp.float32),
        interpret=_INTERPRET,
    )(h, stats, g, b, Wm1, bm1, Wm2, bm2)


NWK = 32
NPW = 320
NPAD = NWK * NPW
QE = 256
EP = E + 2048
FC = 64
NCH = TF // FC
_BIG = 3.0e38

def _get_mesh():
    return plsc.VectorSubcoreMesh(core_axis_name="c", subcore_axis_name="s",
                                  num_cores=2, num_subcores=16)


def _wid():
    return lax.axis_index("s") * 2 + lax.axis_index("c")


def _vgather(x, idx):
    dn = lax.GatherDimensionNumbers(offset_dims=(), collapsed_slice_dims=(0,),
                                    start_index_map=(0,))
    return lax.gather(x, idx[:, None], dimension_numbers=dn, slice_sizes=(1,),
                      mode=lax.GatherScatterMode.PROMISE_IN_BOUNDS)


_CH = 2000


def _prepass_body(srce_ref, dste_ref, packed_ref, nq_ref, src_v, dst_v, stage_v, tmp_v):
    w = _wid()
    lo = w * NPW
    hi = lo + NPW
    dummy = jnp.full((16,), NPW << 16, jnp.int32)

    def chunk_body(ci, carry):
        cbase = pl.multiple_of(ci * _CH, 8)
        pltpu.sync_copy(srce_ref.at[pl.ds(cbase, _CH)], src_v)
        pltpu.sync_copy(dste_ref.at[pl.ds(cbase, _CH)], dst_v)

        def grp(gi, c2):
            cnt, off = c2
            s = src_v[pl.ds(gi * 16, 16)]
            dvec = dst_v[pl.ds(gi * 16, 16)]
            m = (dvec >= lo) & (dvec < hi)
            pk = s | ((dvec - lo) << 16)
            mi = m.astype(jnp.int32)
            pos = cnt + plsc.cumsum(mi) - 1
            plsc.store_scatter(stage_v, [pos], pk, mask=m)
            cnt = cnt + jnp.sum(mi)

            def do_flush(co):
                c, o = co
                pltpu.sync_copy(stage_v.at[pl.ds(0, 1024)],
                                packed_ref.at[pl.ds(pl.multiple_of(w * EP + o, 8), 1024)])
                stage_v[pl.ds(0, 16)] = stage_v[pl.ds(1024, 16)]
                stage_v[pl.ds(16, 16)] = stage_v[pl.ds(1040, 16)]
                return (c - 1024, o + 1024)

            return lax.cond(cnt >= 1024, do_flush, lambda co: co, (cnt, off))

        return lax.fori_loop(0, _CH // 16, grp, carry)

    cnt, off = lax.fori_loop(0, E // _CH, chunk_body, (0, 0))

    stage_v[pl.ds(cnt, 16)] = dummy
    cnt = (cnt + 15) & -16

    def pad16(c):
        stage_v[pl.ds(c, 16)] = dummy
        return c + 16

    cnt = lax.while_loop(lambda c: lax.rem(c, 2 * QE) != 0, pad16, cnt)

    def fflush(qi, _):
        pltpu.sync_copy(stage_v.at[pl.ds(pl.multiple_of(qi * QE, 8), QE)],
                        packed_ref.at[pl.ds(pl.multiple_of(w * EP + off + qi * QE, 8), QE)])
        return 0

    lax.fori_loop(0, cnt // QE, fflush, 0)
    tmp_v[...] = jnp.broadcast_to((off + cnt) // QE, (16,)).astype(jnp.int32)
    pltpu.sync_copy(tmp_v, nq_ref.at[pl.ds(pl.multiple_of(w * 16, 8), 16)])


def _prepass(srce, dste):
    f = pl.kernel(
        _prepass_body,
        out_type=[jax.ShapeDtypeStruct((NWK * EP,), jnp.int32),
                  jax.ShapeDtypeStruct((NWK * 16,), jnp.int32)],
        mesh=_get_mesh(),
        compiler_params=pltpu.CompilerParams(needs_layout_passes=False,
                                             use_tc_tiling_on_sc=False),
        scratch_types=[pltpu.VMEM((_CH,), jnp.int32),
                       pltpu.VMEM((_CH,), jnp.int32),
                       pltpu.VMEM((1600,), jnp.int32),
                       pltpu.VMEM((16,), jnp.int32)],
    )
    return f(srce, dste)


def _stats_body(packed_ref, nq_ref, tab_ref, initz_ref, initmn_ref, initmx_ref,
                s3_ref, q3_ref, mn3_ref, mx3_ref, cnt_ref,
                nq_v, pk_v, srcb0, srcb1, dlb0, dlb1, rows0, rows1,
                accS, accQ, accMN, accMX, accC, sem_p, sem_g0, sem_g1):
    w = _wid()
    iota = jnp.arange(16, dtype=jnp.int32)
    onehot = (iota == 0).astype(jnp.float32)
    pltpu.sync_copy(nq_ref.at[pl.ds(pl.multiple_of(w * 16, 8), 16)], nq_v)
    nq = jnp.max(nq_v[...])
    npairs = nq // 2

    def pk_src(qi):
        return packed_ref.at[pl.ds(pl.multiple_of(w * EP + qi * QE, 8), QE)]

    def unpack(srcb, dlb):
        for k in range(QE // 16):
            pk = pk_v[pl.ds(k * 16, 16)]
            srcb[pl.ds(k * 16, 16)] = pk & 0xFFFF
            dlb[pl.ds(k * 16, 16)] = pk >> 16

    def proc(rows_r, dlb_r):
        def grp(g, ___):
            dl_v = dlb_r[pl.ds(g * 16, 16)]
            for l in range(16):
                dl = _vgather(dl_v, jnp.full((16,), l, jnp.int32))
                dlF = dl * FC + iota
                j = g * 16 + l
                plsc.addupdate_scatter(accC, [dl + iota], onehot)
                for gg in range(FC // 16):
                    v = rows_r[j, pl.ds(gg * 16, 16)]
                    idx = dlF + (gg * 16)
                    plsc.addupdate_scatter(accS, [idx], v)
                    plsc.addupdate_scatter(accQ, [idx], v * v)
                    mn = plsc.load_gather(accMN, [idx])
                    plsc.store_scatter(accMN, [idx], jnp.minimum(mn, v))
                    mx = plsc.load_gather(accMX, [idx])
                    plsc.store_scatter(accMX, [idx], jnp.maximum(mx, v))
            return 0

        lax.fori_loop(0, QE // 16, grp, 0)

    def chunk(c, _):
        pltpu.sync_copy(initz_ref, accS)
        pltpu.sync_copy(initz_ref, accQ)
        pltpu.sync_copy(initmn_ref, accMN)
        pltpu.sync_copy(initmx_ref, accMX)

        @pl.when(c == 0)
        def _():
            def initc(i, __):
                accC[pl.ds(i * 16, 16)] = jnp.zeros((16,), jnp.float32)
                return 0
            lax.fori_loop(0, (NPW + 16) // 16, initc, 0)

        @pl.when(npairs > 0)
        def _():
            pltpu.async_copy(pk_src(0), pk_v, sem_p)

            def pair(p, __):
                q0 = 2 * p
                pltpu.make_async_copy(pk_src(q0), pk_v, sem_p).wait()
                unpack(srcb0, dlb0)
                pltpu.async_copy(tab_ref.at[c].at[srcb0], rows0, sem_g0)
                pltpu.async_copy(pk_src(q0 + 1), pk_v, sem_p)

                @pl.when(p > 0)
                def _():
                    pltpu.make_async_copy(tab_ref.at[c].at[srcb1], rows1,
                                          sem_g1).wait()
                    proc(rows1, dlb1)

                pltpu.make_async_copy(pk_src(q0 + 1), pk_v, sem_p).wait()
                unpack(srcb1, dlb1)
                pltpu.async_copy(tab_ref.at[c].at[srcb1], rows1, sem_g1)

                @pl.when(p + 1 < npairs)
                def _():
                    pltpu.async_copy(pk_src(q0 + 2), pk_v, sem_p)

                pltpu.make_async_copy(tab_ref.at[c].at[srcb0], rows0,
                                      sem_g0).wait()
                proc(rows0, dlb0)
                return 0

            lax.fori_loop(0, npairs, pair, 0)
            pltpu.make_async_copy(tab_ref.at[c].at[srcb1], rows1, sem_g1).wait()
            proc(rows1, dlb1)

        cstride = NPAD * FC
        obase = pl.multiple_of(c * cstride + w * NPW * FC, 8)
        pltpu.sync_copy(accS.at[pl.ds(0, NPW * FC)], s3_ref.at[pl.ds(obase, NPW * FC)])
        pltpu.sync_copy(accQ.at[pl.ds(0, NPW * FC)], q3_ref.at[pl.ds(obase, NPW * FC)])
        pltpu.sync_copy(accMN.at[pl.ds(0, NPW * FC)], mn3_ref.at[pl.ds(obase, NPW * FC)])
        pltpu.sync_copy(accMX.at[pl.ds(0, NPW * FC)], mx3_ref.at[pl.ds(obase, NPW * FC)])

        @pl.when(c == NCH - 1)
        def _():
            def scale(i, __):
                accC[pl.ds(i * 16, 16)] = accC[pl.ds(i * 16, 16)] * (1.0 / NCH)
                return 0
            lax.fori_loop(0, NPW // 16, scale, 0)
            pltpu.sync_copy(accC.at[pl.ds(0, NPW)], cnt_ref.at[pl.ds(pl.multiple_of(w * NPW, 8), NPW)])

        return 0

    lax.fori_loop(0, NCH, chunk, 0)


def _stats(packed, nqarr, tab, initz, initmn, initmx):
    f = pl.kernel(
        _stats_body,
        out_type=[jax.ShapeDtypeStruct((NCH * NPAD * FC,), jnp.float32),
                  jax.ShapeDtypeStruct((NCH * NPAD * FC,), jnp.float32),
                  jax.ShapeDtypeStruct((NCH * NPAD * FC,), jnp.float32),
                  jax.ShapeDtypeStruct((NCH * NPAD * FC,), jnp.float32),
                  jax.ShapeDtypeStruct((NWK * NPW,), jnp.float32)],
        mesh=_get_mesh(),
        compiler_params=pltpu.CompilerParams(needs_layout_passes=False,
                                             use_tc_tiling_on_sc=False),
        scratch_types=[pltpu.VMEM((16,), jnp.int32),
                       pltpu.VMEM((QE,), jnp.int32),
                       pltpu.VMEM((QE,), jnp.int32),
                       pltpu.VMEM((QE,), jnp.int32),
                       pltpu.VMEM((QE,), jnp.int32),
                       pltpu.VMEM((QE,), jnp.int32),
                       pltpu.VMEM((QE, FC), jnp.float32),
                       pltpu.VMEM((QE, FC), jnp.float32),
                       pltpu.VMEM(((NPW + 1) * FC,), jnp.float32),
                       pltpu.VMEM(((NPW + 1) * FC,), jnp.float32),
                       pltpu.VMEM(((NPW + 1) * FC,), jnp.float32),
                       pltpu.VMEM(((NPW + 1) * FC,), jnp.float32),
                       pltpu.VMEM((NPW + 16, ), jnp.float32),
                       pltpu.SemaphoreType.DMA,
                       pltpu.SemaphoreType.DMA,
                       pltpu.SemaphoreType.DMA],
    )
    return f(packed, nqarr, tab, initz, initmn, initmx)


def _edge_phase_sc(tab, packed, nqarr, inits):
    S3f, Q3f, MN3f, MX3f, cnt1 = _stats(packed, nqarr, tab, *inits)
    sh = (NCH, NPAD, FC)
    cnt = cnt1[:N].reshape(N, 1)
    return cnt, S3f.reshape(sh), Q3f.reshape(sh), MN3f.reshape(sh), MX3f.reshape(sh)


def _prep_weights(Wpre, bpre, Wpost, bpost):
    Wt = Wpre[:, :F, :].transpose(1, 0, 2).reshape(F, TF)
    Wb = Wpre[:, F:, :].transpose(1, 0, 2).reshape(F, TF)
    bpre2 = bpre.reshape(1, TF)
    bpb = bpost.reshape(1, T * FOUT)
    return Wt, Wb, bpre2, Wpost, bpb


def kernel(x, edge_index, Wpre1, bpre1, Wpost1, bpost1, Wlin1, blin1,
           Wpre2, bpre2, Wpost2, bpost2, Wlin2, blin2,
           g1, b1, g2, b2, Wm1, bm1, Wm2, bm2):
    src = edge_index[0]
    dst = edge_index[1]

    Wt1, Wb1, bp1, Wp1, bpb1 = _prep_weights(Wpre1, bpre1, Wpost1, bpost1)
    Wt2, Wb2, bp2, Wp2, bpb2 = _prep_weights(Wpre2, bpre2, Wpost2, bpost2)

    packed, nqarr = _prepass(src, dst)

    A1, B1 = _pre_call(x, Wt1, Wb1, bp1)
    inits = (jnp.zeros(((NPW + 1) * FC,), jnp.float32),
             jnp.full(((NPW + 1) * FC,), _BIG, jnp.float32),
             jnp.full(((NPW + 1) * FC,), -_BIG, jnp.float32))
    cnt, SB, QB, mnB, mxB = _edge_phase_sc(B1, packed, nqarr, inits)
    h1, stats1 = _post_call(x, cnt, SB, QB, mnB, mxB, A1,
                            Wp1, bpb1, Wlin1, blin1.reshape(1, F))

    xn2, A2, B2 = _pre_bn_call(h1, stats1, g1.reshape(1, F), b1.reshape(1, F), Wt2, Wb2, bp2)
    cnt2, SB2, QB2, mnB2, mxB2 = _edge_phase_sc(B2, packed, nqarr, inits)
    h2, stats2 = _post_call(xn2, cnt2, SB2, QB2, mnB2, mxB2, A2,
                            Wp2, bpb2, Wlin2, blin2.reshape(1, F))

    return _head_call(h2, stats2, g2.reshape(1, F), b2.reshape(1, F),
                      Wm1, bm1.reshape(1, F), Wm2, bm2.reshape(1, F))


# speedup vs baseline: 45.9063x; 1.8037x over previous
import functools
import math

import jax
import jax.numpy as jnp
from jax import lax
from jax.experimental import pallas as pl
from jax.experimental.pallas import tpu as pltpu
from jax.experimental.pallas import tpu_sc as plsc

N = 10000
E = 320000
F = 128
T = 4
TF = T * F
FOUT = 32
AVG_LOG = float(math.log(33.0))
NB = 1000
GRID_N = N // NB

_INTERPRET = False


def _pre_body(xn_ref, wt_ref, wb_ref, bpre_ref, a_ref, b3_ref):
    xn = xn_ref[...]
    a_ref[...] = jnp.dot(xn, wt_ref[...], preferred_element_type=jnp.float32)
    b = jnp.dot(xn, wb_ref[...], preferred_element_type=jnp.float32) + bpre_ref[...]
    for c in range(NCH):
        b3_ref[c] = b[:, c * FC:(c + 1) * FC]


def _pre_bn_body(h_ref, stats_ref, g_ref, b_ref, wt_ref, wb_ref, bpre_ref,
                 xn_ref, a_ref, b3_ref):
    s = stats_ref[...]
    m = s[0:1, :] * (1.0 / N)
    v = s[1:2, :] * (1.0 / N) - m * m
    inv = jax.lax.rsqrt(v + 1e-5)
    xn = jnp.maximum((h_ref[...] - m) * inv * g_ref[...] + b_ref[...], 0.0)
    xn_ref[...] = xn
    a_ref[...] = jnp.dot(xn, wt_ref[...], preferred_element_type=jnp.float32)
    b = jnp.dot(xn, wb_ref[...], preferred_element_type=jnp.float32) + bpre_ref[...]
    for c in range(NCH):
        b3_ref[c] = b[:, c * FC:(c + 1) * FC]


def _post_body(xn_ref, cnt_ref, sb_ref, qb_ref, mnb_ref, mxb_ref, a_ref,
               wp_ref, bpb_ref, wlin_ref, blin_ref,
               h_ref, stats_ref, acc_ref):
    i = pl.program_id(0)
    xn = xn_ref[...]
    a = a_ref[...]
    cnt = cnt_ref[...]
    c1 = jnp.maximum(cnt, 1.0)
    sb = jnp.concatenate([sb_ref[c] for c in range(NCH)], axis=-1)
    qb = jnp.concatenate([qb_ref[c] for c in range(NCH)], axis=-1)
    mnb = jnp.concatenate([mnb_ref[c] for c in range(NCH)], axis=-1)
    mxb = jnp.concatenate([mxb_ref[c] for c in range(NCH)], axis=-1)
    s = cnt * a + sb
    mean = s / c1
    ssq = cnt * a * a + 2.0 * a * sb + qb
    var = ssq / c1 - mean * mean
    std = jnp.sqrt(jnp.maximum(var, 0.0) + 1e-5)
    has = cnt > 0.0
    mn = jnp.where(has, a + mnb, 0.0)
    mx = jnp.where(has, a + mxb, 0.0)
    lg = jnp.log(c1 + 1.0)
    amp_s = lg * (1.0 / AVG_LOG)
    att_s = AVG_LOG / lg
    outs = []
    for t in range(T):
        sl = slice(t * F, (t + 1) * F)
        agg_t = jnp.concatenate([mean[:, sl], mn[:, sl], mx[:, sl], std[:, sl]], axis=1)
        pin = jnp.concatenate([xn, amp_s * agg_t, att_s * agg_t, agg_t], axis=1)
        outs.append(jnp.dot(pin, wp_ref[t], preferred_element_type=jnp.float32))
    flat = jnp.concatenate(outs, axis=1) + bpb_ref[...]
    h = jnp.dot(flat, wlin_ref[...], preferred_element_type=jnp.float32) + blin_ref[...]
    h_ref[...] = h

    @pl.when(i == 0)
    def _():
        acc_ref[...] = jnp.zeros_like(acc_ref)

    acc_ref[0:1, :] += jnp.sum(h, axis=0, keepdims=True)
    acc_ref[1:2, :] += jnp.sum(h * h, axis=0, keepdims=True)

    @pl.when(i == GRID_N - 1)
    def _():
        stats_ref[...] = acc_ref[...]


def _head_body(h_ref, stats_ref, g_ref, b_ref, wm1_ref, bm1_ref, wm2_ref, bm2_ref, out_ref):
    s = stats_ref[...]
    m = s[0:1, :] * (1.0 / N)
    v = s[1:2, :] * (1.0 / N) - m * m
    inv = jax.lax.rsqrt(v + 1e-5)
    xn = jnp.maximum((h_ref[...] - m) * inv * g_ref[...] + b_ref[...], 0.0)
    y = jnp.maximum(jnp.dot(xn, wm1_ref[...], preferred_element_type=jnp.float32) + bm1_ref[...], 0.0)
    out_ref[...] = jnp.dot(y, wm2_ref[...], preferred_element_type=jnp.float32) + bm2_ref[...]


def _row_spec(cols):
    return pl.BlockSpec((NB, cols), lambda i: (i, 0))


def _chunk_spec():
    return pl.BlockSpec((NCH, NB, FC), lambda i: (0, i, 0))


def _full_spec(shape):
    nd = len(shape)
    return pl.BlockSpec(shape, lambda i: (0,) * nd)


def _pre_call(xn, Wt, Wb, bpre2):
    return pl.pallas_call(
        _pre_body,
        grid=(GRID_N,),
        in_specs=[_row_spec(F), _full_spec((F, TF)), _full_spec((F, TF)), _full_spec((1, TF))],
        out_specs=[_row_spec(TF), _chunk_spec()],
        out_shape=[jax.ShapeDtypeStruct((N, TF), jnp.float32),
                   jax.ShapeDtypeStruct((NCH, N, FC), jnp.float32)],
        interpret=_INTERPRET,
    )(xn, Wt, Wb, bpre2)


def _pre_bn_call(h, stats, g, b, Wt, Wb, bpre2):
    return pl.pallas_call(
        _pre_bn_body,
        grid=(GRID_N,),
        in_specs=[_row_spec(F), _full_spec((2, F)), _full_spec((1, F)), _full_spec((1, F)),
                  _full_spec((F, TF)), _full_spec((F, TF)), _full_spec((1, TF))],
        out_specs=[_row_spec(F), _row_spec(TF), _chunk_spec()],
        out_shape=[jax.ShapeDtypeStruct((N, F), jnp.float32),
                   jax.ShapeDtypeStruct((N, TF), jnp.float32),
                   jax.ShapeDtypeStruct((NCH, N, FC), jnp.float32)],
        interpret=_INTERPRET,
    )(h, stats, g, b, Wt, Wb, bpre2)


def _post_call(xn, cnt, SB, QB, mnB, mxB, A, Wp, bpb, Wlin, blin):
    return pl.pallas_call(
        _post_body,
        grid=(GRID_N,),
        in_specs=[_row_spec(F), _row_spec(1), _chunk_spec(), _chunk_spec(),
                  _chunk_spec(), _chunk_spec(), _row_spec(TF),
                  _full_spec((T, 13 * F, FOUT)),
                  _full_spec((1, F)), _full_spec((F, F)), _full_spec((1, F))],
        out_specs=[_row_spec(F), _full_spec((2, F))],
        out_shape=[jax.ShapeDtypeStruct((N, F), jnp.float32),
                   jax.ShapeDtypeStruct((2, F), jnp.float32)],
        scratch_shapes=[pltpu.VMEM((2, F), jnp.float32)],
        interpret=_INTERPRET,
    )(xn, cnt, SB, QB, mnB, mxB, A, Wp, bpb, Wlin, blin)


def _head_call(h, stats, g, b, Wm1, bm1, Wm2, bm2):
    return pl.pallas_call(
        _head_body,
        grid=(GRID_N,),
        in_specs=[_row_spec(F), _full_spec((2, F)), _full_spec((1, F)), _full_spec((1, F)),
                  _full_spec((F, F)), _full_spec((1, F)), _full_spec((F, F)), _full_spec((1, F))],
        out_specs=_row_spec(F),
        out_shape=jax.ShapeDtypeStruct((N, F), jnp.float32),
        interpret=_INTERPRET,
    )(h, stats, g, b, Wm1, bm1, Wm2, bm2)


NWK = 32
NPW = 320
NPAD = NWK * NPW
QE = 256
EP = E + 2048
FC = 64
NCH = TF // FC
_BIG = 3.0e38

def _get_mesh():
    return plsc.VectorSubcoreMesh(core_axis_name="c", subcore_axis_name="s",
                                  num_cores=2, num_subcores=16)


def _wid():
    return lax.axis_index("s") * 2 + lax.axis_index("c")


def _vgather(x, idx):
    dn = lax.GatherDimensionNumbers(offset_dims=(), collapsed_slice_dims=(0,),
                                    start_index_map=(0,))
    return lax.gather(x, idx[:, None], dimension_numbers=dn, slice_sizes=(1,),
                      mode=lax.GatherScatterMode.PROMISE_IN_BOUNDS)


_CH = 2000


def _prepass_body(srce_ref, dste_ref, packed_ref, nq_ref, cnt_ref,
                  src_v, dst_v, stage_v, tmp_v, accC):
    w = _wid()
    lo = w * NPW
    hi = lo + NPW
    dummy = jnp.full((16,), NPW << 16, jnp.int32)
    ones16 = jnp.ones((16,), jnp.float32)

    def initc(i, __):
        accC[pl.ds(i * 16, 16)] = jnp.zeros((16,), jnp.float32)
        return 0

    lax.fori_loop(0, (NPW + 16) // 16, initc, 0)

    def chunk_body(ci, carry):
        cbase = pl.multiple_of(ci * _CH, 8)
        pltpu.sync_copy(srce_ref.at[pl.ds(cbase, _CH)], src_v)
        pltpu.sync_copy(dste_ref.at[pl.ds(cbase, _CH)], dst_v)

        def grp(gi, c2):
            cnt, off = c2
            s = src_v[pl.ds(gi * 16, 16)]
            dvec = dst_v[pl.ds(gi * 16, 16)]
            m = (dvec >= lo) & (dvec < hi)
            pk = s | ((dvec - lo) << 16)
            mi = m.astype(jnp.int32)
            pos = cnt + plsc.cumsum(mi) - 1
            plsc.store_scatter(stage_v, [pos], pk, mask=m)
            plsc.addupdate_scatter(accC, [jnp.where(m, dvec - lo, NPW)], ones16,
                                   mask=m)
            cnt = cnt + jnp.sum(mi)

            def do_flush(co):
                c, o = co
                pltpu.sync_copy(stage_v.at[pl.ds(0, 1024)],
                                packed_ref.at[pl.ds(pl.multiple_of(w * EP + o, 8), 1024)])
                stage_v[pl.ds(0, 16)] = stage_v[pl.ds(1024, 16)]
                stage_v[pl.ds(16, 16)] = stage_v[pl.ds(1040, 16)]
                return (c - 1024, o + 1024)

            return lax.cond(cnt >= 1024, do_flush, lambda co: co, (cnt, off))

        return lax.fori_loop(0, _CH // 16, grp, carry)

    cnt, off = lax.fori_loop(0, E // _CH, chunk_body, (0, 0))

    stage_v[pl.ds(cnt, 16)] = dummy
    cnt = (cnt + 15) & -16

    def pad16(c):
        stage_v[pl.ds(c, 16)] = dummy
        return c + 16

    cnt = lax.while_loop(lambda c: lax.rem(c, 2 * QE) != 0, pad16, cnt)

    def fflush(qi, _):
        pltpu.sync_copy(stage_v.at[pl.ds(pl.multiple_of(qi * QE, 8), QE)],
                        packed_ref.at[pl.ds(pl.multiple_of(w * EP + off + qi * QE, 8), QE)])
        return 0

    lax.fori_loop(0, cnt // QE, fflush, 0)
    tmp_v[...] = jnp.broadcast_to((off + cnt) // QE, (16,)).astype(jnp.int32)
    pltpu.sync_copy(tmp_v, nq_ref.at[pl.ds(pl.multiple_of(w * 16, 8), 16)])
    pltpu.sync_copy(accC.at[pl.ds(0, NPW)],
                    cnt_ref.at[pl.ds(pl.multiple_of(w * NPW, 8), NPW)])


def _prepass(srce, dste):
    f = pl.kernel(
        _prepass_body,
        out_type=[jax.ShapeDtypeStruct((NWK * EP,), jnp.int32),
                  jax.ShapeDtypeStruct((NWK * 16,), jnp.int32),
                  jax.ShapeDtypeStruct((NWK * NPW,), jnp.float32)],
        mesh=_get_mesh(),
        compiler_params=pltpu.CompilerParams(needs_layout_passes=False,
                                             use_tc_tiling_on_sc=False),
        scratch_types=[pltpu.VMEM((_CH,), jnp.int32),
                       pltpu.VMEM((_CH,), jnp.int32),
                       pltpu.VMEM((1600,), jnp.int32),
                       pltpu.VMEM((16,), jnp.int32),
                       pltpu.VMEM((NPW + 16,), jnp.float32)],
    )
    return f(srce, dste)


def _stats_body(packed_ref, nq_ref, tab_ref, initz_ref, initmn_ref, initmx_ref,
                s3_ref, q3_ref, mn3_ref, mx3_ref,
                nq_v, pk_v, srcb0, srcb1, dlb0, dlb1, rows0, rows1,
                accS, accQ, accMN, accMX, sem_p, sem_g0, sem_g1):
    w = _wid()
    pltpu.sync_copy(nq_ref.at[pl.ds(pl.multiple_of(w * 16, 8), 16)], nq_v)
    nq = jnp.max(nq_v[...])
    npairs = nq // 2

    def pk_src(qi):
        return packed_ref.at[pl.ds(pl.multiple_of(w * EP + qi * QE, 8), QE)]

    def unpack(srcb, dlb):
        for k in range(QE // 16):
            pk = pk_v[pl.ds(k * 16, 16)]
            srcb[pl.ds(k * 16, 16)] = pk & 0xFFFF
            dlb[pl.ds(k * 16, 16)] = pk >> 16

    def proc(rows_r, dlb_r):
        def grp(g, ___):
            dl_v = dlb_r[pl.ds(g * 16, 16)]
            for l in range(16):
                base = dl_v[l] * FC
                j = g * 16 + l
                for gg in range(FC // 16):
                    off = base + gg * 16
                    v = rows_r[j, pl.ds(gg * 16, 16)]
                    plsc.addupdate(accS.at[pl.ds(off, 16)], v)
                    plsc.addupdate(accQ.at[pl.ds(off, 16)], v * v)
                    mn = accMN[pl.ds(off, 16)]
                    accMN[pl.ds(off, 16)] = jnp.minimum(mn, v)
                    mx = accMX[pl.ds(off, 16)]
                    accMX[pl.ds(off, 16)] = jnp.maximum(mx, v)
            return 0

        lax.fori_loop(0, QE // 16, grp, 0)

    def chunk(c, _):
        pltpu.sync_copy(initz_ref, accS)
        pltpu.sync_copy(initz_ref, accQ)
        pltpu.sync_copy(initmn_ref, accMN)
        pltpu.sync_copy(initmx_ref, accMX)

        @pl.when(npairs > 0)
        def _():
            pltpu.async_copy(pk_src(0), pk_v, sem_p)

            def pair(p, __):
                q0 = 2 * p
                pltpu.make_async_copy(pk_src(q0), pk_v, sem_p).wait()
                unpack(srcb0, dlb0)
                pltpu.async_copy(tab_ref.at[c].at[srcb0], rows0, sem_g0)
                pltpu.async_copy(pk_src(q0 + 1), pk_v, sem_p)

                @pl.when(p > 0)
                def _():
                    pltpu.make_async_copy(tab_ref.at[c].at[srcb1], rows1,
                                          sem_g1).wait()
                    proc(rows1, dlb1)

                pltpu.make_async_copy(pk_src(q0 + 1), pk_v, sem_p).wait()
                unpack(srcb1, dlb1)
                pltpu.async_copy(tab_ref.at[c].at[srcb1], rows1, sem_g1)

                @pl.when(p + 1 < npairs)
                def _():
                    pltpu.async_copy(pk_src(q0 + 2), pk_v, sem_p)

                pltpu.make_async_copy(tab_ref.at[c].at[srcb0], rows0,
                                      sem_g0).wait()
                proc(rows0, dlb0)
                return 0

            lax.fori_loop(0, npairs, pair, 0)
            pltpu.make_async_copy(tab_ref.at[c].at[srcb1], rows1, sem_g1).wait()
            proc(rows1, dlb1)

        cstride = NPAD * FC
        obase = pl.multiple_of(c * cstride + w * NPW * FC, 8)
        pltpu.sync_copy(accS.at[pl.ds(0, NPW * FC)], s3_ref.at[pl.ds(obase, NPW * FC)])
        pltpu.sync_copy(accQ.at[pl.ds(0, NPW * FC)], q3_ref.at[pl.ds(obase, NPW * FC)])
        pltpu.sync_copy(accMN.at[pl.ds(0, NPW * FC)], mn3_ref.at[pl.ds(obase, NPW * FC)])
        pltpu.sync_copy(accMX.at[pl.ds(0, NPW * FC)], mx3_ref.at[pl.ds(obase, NPW * FC)])

        return 0

    lax.fori_loop(0, NCH, chunk, 0)


def _stats(packed, nqarr, tab, initz, initmn, initmx):
    f = pl.kernel(
        _stats_body,
        out_type=[jax.ShapeDtypeStruct((NCH * NPAD * FC,), jnp.float32),
                  jax.ShapeDtypeStruct((NCH * NPAD * FC,), jnp.float32),
                  jax.ShapeDtypeStruct((NCH * NPAD * FC,), jnp.float32),
                  jax.ShapeDtypeStruct((NCH * NPAD * FC,), jnp.float32)],
        mesh=_get_mesh(),
        compiler_params=pltpu.CompilerParams(needs_layout_passes=False,
                                             use_tc_tiling_on_sc=False),
        scratch_types=[pltpu.VMEM((16,), jnp.int32),
                       pltpu.VMEM((QE,), jnp.int32),
                       pltpu.VMEM((QE,), jnp.int32),
                       pltpu.VMEM((QE,), jnp.int32),
                       pltpu.VMEM((QE,), jnp.int32),
                       pltpu.VMEM((QE,), jnp.int32),
                       pltpu.VMEM((QE, FC), jnp.float32),
                       pltpu.VMEM((QE, FC), jnp.float32),
                       pltpu.VMEM(((NPW + 1) * FC,), jnp.float32),
                       pltpu.VMEM(((NPW + 1) * FC,), jnp.float32),
                       pltpu.VMEM(((NPW + 1) * FC,), jnp.float32),
                       pltpu.VMEM(((NPW + 1) * FC,), jnp.float32),
                       pltpu.SemaphoreType.DMA,
                       pltpu.SemaphoreType.DMA,
                       pltpu.SemaphoreType.DMA],
    )
    return f(packed, nqarr, tab, initz, initmn, initmx)


def _edge_phase_sc(tab, packed, nqarr, inits):
    S3f, Q3f, MN3f, MX3f = _stats(packed, nqarr, tab, *inits)
    sh = (NCH, NPAD, FC)
    return S3f.reshape(sh), Q3f.reshape(sh), MN3f.reshape(sh), MX3f.reshape(sh)


def _prep_weights(Wpre, bpre, Wpost, bpost):
    Wt = Wpre[:, :F, :].transpose(1, 0, 2).reshape(F, TF)
    Wb = Wpre[:, F:, :].transpose(1, 0, 2).reshape(F, TF)
    bpre2 = bpre.reshape(1, TF)
    bpb = bpost.reshape(1, T * FOUT)
    return Wt, Wb, bpre2, Wpost, bpb


def kernel(x, edge_index, Wpre1, bpre1, Wpost1, bpost1, Wlin1, blin1,
           Wpre2, bpre2, Wpost2, bpost2, Wlin2, blin2,
           g1, b1, g2, b2, Wm1, bm1, Wm2, bm2):
    src = edge_index[0]
    dst = edge_index[1]

    Wt1, Wb1, bp1, Wp1, bpb1 = _prep_weights(Wpre1, bpre1, Wpost1, bpost1)
    Wt2, Wb2, bp2, Wp2, bpb2 = _prep_weights(Wpre2, bpre2, Wpost2, bpost2)

    packed, nqarr, cnt1 = _prepass(src, dst)
    cnt = cnt1[:N].reshape(N, 1)

    A1, B1 = _pre_call(x, Wt1, Wb1, bp1)
    inits = (jnp.zeros(((NPW + 1) * FC,), jnp.float32),
             jnp.full(((NPW + 1) * FC,), _BIG, jnp.float32),
             jnp.full(((NPW + 1) * FC,), -_BIG, jnp.float32))
    SB, QB, mnB, mxB = _edge_phase_sc(B1, packed, nqarr, inits)
    h1, stats1 = _post_call(x, cnt, SB, QB, mnB, mxB, A1,
                            Wp1, bpb1, Wlin1, blin1.reshape(1, F))

    xn2, A2, B2 = _pre_bn_call(h1, stats1, g1.reshape(1, F), b1.reshape(1, F), Wt2, Wb2, bp2)
    SB2, QB2, mnB2, mxB2 = _edge_phase_sc(B2, packed, nqarr, inits)
    h2, stats2 = _post_call(xn2, cnt, SB2, QB2, mnB2, mxB2, A2,
                            Wp2, bpb2, Wlin2, blin2.reshape(1, F))

    return _head_call(h2, stats2, g2.reshape(1, F), b2.reshape(1, F),
                      Wm1, bm1.reshape(1, F), Wm2, bm2.reshape(1, F))
